# baseline (speedup 1.0000x reference)
import functools

import jax
import jax.numpy as jnp
from jax import lax
from jax.experimental import pallas as pl
from jax.experimental.pallas import tpu as pltpu
from jax.experimental.pallas import tpu_sc as plsc

BATCH = 4096
EMBED_DIM = 16
ROWS_PER_LINE = 8
LANES = 16
WAVE = 16


def _sc_gather_sum(user, item, behavior, ut_t, it_t, bt, nrows):
    mesh = plsc.VectorSubcoreMesh(core_axis_name="c", subcore_axis_name="s")
    nc, ns = mesh.num_cores, mesh.num_subcores
    nw = nc * ns
    b_per_w = nrows // nw

    @functools.partial(
        pl.kernel,
        out_type=jax.ShapeDtypeStruct((nrows, EMBED_DIM), jnp.float32),
        mesh=mesh,
        scratch_types=[
            pltpu.VMEM((b_per_w + LANES,), jnp.int32),
            pltpu.VMEM((b_per_w + LANES,), jnp.int32),
            pltpu.VMEM((b_per_w,), jnp.int32),
            pltpu.VMEM((WAVE, LANES, 128), jnp.float32),
            pltpu.VMEM((WAVE, LANES, 128), jnp.float32),
            pltpu.VMEM((8, LANES), jnp.float32),
            pltpu.VMEM((b_per_w, EMBED_DIM), jnp.float32),
            pltpu.SemaphoreType.DMA,
        ],
        compiler_params=pltpu.CompilerParams(needs_layout_passes=False),
    )
    def gather_kernel(user_hbm, item_hbm, beh_hbm, ut_hbm, it_hbm, bt_hbm,
                      x_hbm, uidx, iidx, bidx,
                      ublk, iblk, btv, outv, sem):
        wid = lax.axis_index("s") * nc + lax.axis_index("c")
        base = wid * b_per_w
        pltpu.sync_copy(user_hbm.at[pl.ds(base, b_per_w)],
                        uidx.at[pl.ds(0, b_per_w)])
        pltpu.sync_copy(item_hbm.at[pl.ds(base, b_per_w)],
                        iidx.at[pl.ds(0, b_per_w)])
        pltpu.sync_copy(beh_hbm.at[pl.ds(base, b_per_w)], bidx)
        pltpu.sync_copy(bt_hbm, btv)

        lane = lax.iota(jnp.int32, LANES)

        def wave_body(w, carry):
            w0 = w * WAVE

            def fire_body(k, carry2):
                j = w0 + k
                uj = uidx[pl.ds(j, LANES)][0]
                ij = iidx[pl.ds(j, LANES)][0]
                ua = pl.multiple_of(jnp.bitwise_and(uj, ~127), 128)
                ia = pl.multiple_of(jnp.bitwise_and(ij, ~127), 128)
                pltpu.async_copy(ut_hbm.at[:, pl.ds(ua, 128)], ublk.at[k], sem)
                pltpu.async_copy(it_hbm.at[:, pl.ds(ia, 128)], iblk.at[k], sem)
                return carry2

            lax.fori_loop(0, WAVE, fire_body, 0)

            def drain_body(k, carry2):
                pltpu.make_async_copy(
                    ut_hbm.at[:, pl.ds(0, 128)], ublk.at[k], sem).wait()
                pltpu.make_async_copy(
                    it_hbm.at[:, pl.ds(0, 128)], iblk.at[k], sem).wait()
                return carry2

            lax.fori_loop(0, WAVE, drain_body, 0)

            def row_body(k, carry2):
                j = w0 + k
                j16 = jnp.full((LANES,), j, dtype=jnp.int32)
                k16 = jnp.full((LANES,), k, dtype=jnp.int32)
                uo = jnp.bitwise_and(plsc.load_gather(uidx, [j16]), 127)
                io = jnp.bitwise_and(plsc.load_gather(iidx, [j16]), 127)
                br = plsc.load_gather(bidx, [j16])
                u = plsc.load_gather(ublk, [k16, lane, uo])
                iv = plsc.load_gather(iblk, [k16, lane, io])
                bv = plsc.load_gather(btv, [br, lane])
                outv[j, :] = u + iv + bv
                return carry2

            lax.fori_loop(0, WAVE, row_body, 0)
            return carry

        lax.fori_loop(0, b_per_w // WAVE, wave_body, 0)
        pltpu.sync_copy(outv, x_hbm.at[pl.ds(base, b_per_w)])

    return gather_kernel(user, item, behavior, ut_t, it_t, bt)


def _tc_gcn_partial(adj, x, kcols, kblock, block_m=512):
    def body(adj_ref, x_ref, out_ref):
        out_ref[...] = jnp.dot(adj_ref[...], x_ref[...],
                               preferred_element_type=jnp.float32)

    grid = (BATCH // block_m,)
    return pl.pallas_call(
        body,
        grid=grid,
        in_specs=[
            pl.BlockSpec((block_m, kcols), lambda i: (i, kblock)),
            pl.BlockSpec((kcols, EMBED_DIM), lambda i: (0, 0)),
        ],
        out_specs=pl.BlockSpec((block_m, EMBED_DIM), lambda i: (i, 0)),
        out_shape=jax.ShapeDtypeStruct((BATCH, EMBED_DIM), jnp.float32),
    )(adj, x)


def _tc_gcn_final(adj, x, acc, w_t, b2d, kcols, kblock, block_m=512):
    def body(adj_ref, x_ref, acc_ref, wt_ref, b_ref, out_ref):
        s = acc_ref[...] + jnp.dot(adj_ref[...], x_ref[...],
                                   preferred_element_type=jnp.float32)
        out_ref[...] = jnp.dot(s, wt_ref[...],
                               preferred_element_type=jnp.float32) + b_ref[...]

    grid = (BATCH // block_m,)
    return pl.pallas_call(
        body,
        grid=grid,
        in_specs=[
            pl.BlockSpec((block_m, kcols), lambda i: (i, kblock)),
            pl.BlockSpec((kcols, EMBED_DIM), lambda i: (0, 0)),
            pl.BlockSpec((block_m, EMBED_DIM), lambda i: (i, 0)),
            pl.BlockSpec((EMBED_DIM, EMBED_DIM), lambda i: (0, 0)),
            pl.BlockSpec((1, EMBED_DIM), lambda i: (0, 0)),
        ],
        out_specs=pl.BlockSpec((block_m, EMBED_DIM), lambda i: (i, 0)),
        out_shape=jax.ShapeDtypeStruct((BATCH, EMBED_DIM), jnp.float32),
    )(adj, x, acc, w_t, b2d)


def kernel(user, item, behavior, adj, user_table, item_table, behavior_table,
           W, b):
    ut_t, it_t = user_table.T, item_table.T
    half = BATCH // 2
    x_a = _sc_gather_sum(user[:half], item[:half], behavior[:half],
                         ut_t, it_t, behavior_table, half)
    x_b = _sc_gather_sum(user[half:], item[half:], behavior[half:],
                         ut_t, it_t, behavior_table, half)
    acc = _tc_gcn_partial(adj, x_a, half, 0)
    return _tc_gcn_final(adj, x_b, acc, W.T, b.reshape(1, EMBED_DIM), half, 1)


# speedup vs baseline: 1.0809x; 1.0809x over previous
import functools

import jax
import jax.numpy as jnp
from jax import lax
from jax.experimental import pallas as pl
from jax.experimental.pallas import tpu as pltpu
from jax.experimental.pallas import tpu_sc as plsc

BATCH = 4096
EMBED_DIM = 16
ROWS_PER_LINE = 8
LANES = 16
WAVE = 16


def _sc_gather_sum(user, item, behavior, ut_t, it_t, bt, nrows):
    mesh = plsc.VectorSubcoreMesh(core_axis_name="c", subcore_axis_name="s")
    nc, ns = mesh.num_cores, mesh.num_subcores
    nw = nc * ns
    b_per_w = nrows // nw

    @functools.partial(
        pl.kernel,
        out_type=jax.ShapeDtypeStruct((nrows, EMBED_DIM), jnp.float32),
        mesh=mesh,
        scratch_types=[
            pltpu.VMEM((b_per_w + LANES,), jnp.int32),
            pltpu.VMEM((b_per_w + LANES,), jnp.int32),
            pltpu.VMEM((b_per_w,), jnp.int32),
            pltpu.VMEM((WAVE, LANES, 128), jnp.float32),
            pltpu.VMEM((WAVE, LANES, 128), jnp.float32),
            pltpu.VMEM((8, LANES), jnp.float32),
            pltpu.VMEM((b_per_w, EMBED_DIM), jnp.float32),
            pltpu.SemaphoreType.DMA,
        ],
        compiler_params=pltpu.CompilerParams(needs_layout_passes=False,
                                             skip_device_barrier=True),
    )
    def gather_kernel(user_hbm, item_hbm, beh_hbm, ut_hbm, it_hbm, bt_hbm,
                      x_hbm, uidx, iidx, bidx,
                      ublk, iblk, btv, outv, sem):
        wid = lax.axis_index("s") * nc + lax.axis_index("c")
        base = wid * b_per_w
        c1 = pltpu.async_copy(user_hbm.at[pl.ds(base, b_per_w)],
                              uidx.at[pl.ds(0, b_per_w)], sem)
        c2 = pltpu.async_copy(item_hbm.at[pl.ds(base, b_per_w)],
                              iidx.at[pl.ds(0, b_per_w)], sem)
        c3 = pltpu.async_copy(beh_hbm.at[pl.ds(base, b_per_w)], bidx, sem)
        c4 = pltpu.async_copy(bt_hbm, btv, sem)
        c1.wait()
        c2.wait()
        c3.wait()
        c4.wait()

        lane = lax.iota(jnp.int32, LANES)

        def wave_body(w, carry):
            w0 = w * WAVE

            def fire_body(k, carry2):
                j = w0 + k
                uj = uidx[pl.ds(j, LANES)][0]
                ij = iidx[pl.ds(j, LANES)][0]
                ua = pl.multiple_of(jnp.bitwise_and(uj, ~127), 128)
                ia = pl.multiple_of(jnp.bitwise_and(ij, ~127), 128)
                pltpu.async_copy(ut_hbm.at[:, pl.ds(ua, 128)], ublk.at[k], sem)
                pltpu.async_copy(it_hbm.at[:, pl.ds(ia, 128)], iblk.at[k], sem)
                return carry2

            lax.fori_loop(0, WAVE, fire_body, 0)

            def drain_body(k, carry2):
                pltpu.make_async_copy(
                    ut_hbm.at[:, pl.ds(0, 128)], ublk.at[k], sem).wait()
                pltpu.make_async_copy(
                    it_hbm.at[:, pl.ds(0, 128)], iblk.at[k], sem).wait()
                return carry2

            lax.fori_loop(0, WAVE, drain_body, 0)

            def row_body(k, carry2):
                j = w0 + k
                j16 = jnp.full((LANES,), j, dtype=jnp.int32)
                k16 = jnp.full((LANES,), k, dtype=jnp.int32)
                uo = jnp.bitwise_and(plsc.load_gather(uidx, [j16]), 127)
                io = jnp.bitwise_and(plsc.load_gather(iidx, [j16]), 127)
                br = plsc.load_gather(bidx, [j16])
                u = plsc.load_gather(ublk, [k16, lane, uo])
                iv = plsc.load_gather(iblk, [k16, lane, io])
                bv = plsc.load_gather(btv, [br, lane])
                outv[j, :] = u + iv + bv
                return carry2

            lax.fori_loop(0, WAVE, row_body, 0)
            return carry

        lax.fori_loop(0, b_per_w // WAVE, wave_body, 0)
        pltpu.sync_copy(outv, x_hbm.at[pl.ds(base, b_per_w)])

    return gather_kernel(user, item, behavior, ut_t, it_t, bt)


def _tc_gcn(adj, x, w_t, b2d, block_m=512):
    def body(adj_ref, x_ref, wt_ref, b_ref, out_ref):
        acc = jnp.dot(adj_ref[...], x_ref[...],
                      preferred_element_type=jnp.float32)
        out_ref[...] = jnp.dot(acc, wt_ref[...],
                               preferred_element_type=jnp.float32) + b_ref[...]

    grid = (BATCH // block_m,)
    return pl.pallas_call(
        body,
        grid=grid,
        in_specs=[
            pl.BlockSpec((block_m, BATCH), lambda i: (i, 0)),
            pl.BlockSpec((BATCH, EMBED_DIM), lambda i: (0, 0)),
            pl.BlockSpec((EMBED_DIM, EMBED_DIM), lambda i: (0, 0)),
            pl.BlockSpec((1, EMBED_DIM), lambda i: (0, 0)),
        ],
        out_specs=pl.BlockSpec((block_m, EMBED_DIM), lambda i: (i, 0)),
        out_shape=jax.ShapeDtypeStruct((BATCH, EMBED_DIM), jnp.float32),
    )(adj, x, w_t, b2d)


def kernel(user, item, behavior, adj, user_table, item_table, behavior_table,
           W, b):
    x = _sc_gather_sum(user, item, behavior, user_table.T, item_table.T,
                       behavior_table, BATCH)
    return _tc_gcn(adj, x, W.T, b.reshape(1, EMBED_DIM))
